# baseline (speedup 1.0000x reference)
import functools

import jax
import jax.numpy as jnp
from jax import lax
from jax.experimental import pallas as pl
from jax.experimental.pallas import tpu as pltpu
from jax.experimental.pallas import tpu_sc as plsc

B = 16384
SD = 32
TD = 32
L = 16
NC = 2
NS = 16
NW = NC * NS
PER_W = B // NW
C = 128

_TWO_PI = 6.283185307179586
_INV_TWO_PI = 0.15915494309189535
_MAGIC = 12582912.0


def _sinp(x):
    k = (x * _INV_TWO_PI + _MAGIC) - _MAGIC
    r = x - _TWO_PI * k
    r2 = r * r
    p = jnp.float32(-2.505210838544172e-08)
    p = p * r2 + jnp.float32(2.755731922398589e-06)
    p = p * r2 + jnp.float32(-0.0001984126984126984)
    p = p * r2 + jnp.float32(0.008333333333333333)
    p = p * r2 + jnp.float32(-0.16666666666666666)
    p = p * r2 + jnp.float32(1.0)
    return r * p


def _make_sc_kernel():
    mesh = plsc.VectorSubcoreMesh(core_axis_name="c", subcore_axis_name="s")

    scratch = [
        pltpu.VMEM((C,), jnp.int32),
        pltpu.VMEM((C,), jnp.int32),
        pltpu.VMEM((C,), jnp.int32),
        pltpu.VMEM((C, L), jnp.float32),
        pltpu.VMEM((C, L), jnp.float32),
        pltpu.VMEM((C, L), jnp.float32),
        pltpu.VMEM((C, SD + TD), jnp.float32),
        pltpu.VMEM((C, SD + TD), jnp.float32),
    ] + [pltpu.VMEM((C, TD), jnp.float32) for _ in range(18)] + [
        pltpu.VMEM((C * L,), jnp.float32),
        pltpu.VMEM((C,), jnp.float32),
        pltpu.SemaphoreType.DMA,
    ]

    @functools.partial(
        pl.kernel,
        out_type=jax.ShapeDtypeStruct((B,), jnp.float32),
        mesh=mesh,
        scratch_types=scratch,
        compiler_params=pltpu.CompilerParams(use_tc_tiling_on_sc=False),
    )
    def sck(heads, rels, tails, years_b, months_b, days_b,
            ent_h, ent_t, rel_f, rel_i,
            yfh, yft, mfh, mft, dfh, dft,
            yph, ypt, mph, mpt, dph, dpt,
            yah, yat, mah, mat, dah, dat,
            out,
            idx_h, idx_t, idx_r, ty, tm, td, rf_v, ri_v,
            *rest):
        tbufs = rest[:18]
        acc, out_v, sem = rest[18], rest[19], rest[20]

        wid = lax.axis_index("s") * NC + lax.axis_index("c")
        h_tables = (yfh, yph, yah, mfh, mph, mah, dfh, dph, dah)
        t_tables = (yft, ypt, yat, mft, mpt, mat, dft, dpt, dat)
        rows0 = lax.iota(jnp.int32, L)

        def chunk_body(j, carry):
            base = wid * PER_W + j * C

            pltpu.sync_copy(heads.at[pl.ds(base, C)], idx_h)
            pltpu.sync_copy(tails.at[pl.ds(base, C)], idx_t)
            pltpu.sync_copy(rels.at[pl.ds(base, C)], idx_r)
            pltpu.sync_copy(years_b.at[pl.ds(base, C)], ty)
            pltpu.sync_copy(months_b.at[pl.ds(base, C)], tm)
            pltpu.sync_copy(days_b.at[pl.ds(base, C)], td)

            cps = [
                pltpu.async_copy(ent_h.at[idx_h], tbufs[0], sem),
                pltpu.async_copy(ent_t.at[idx_t], tbufs[1], sem),
                pltpu.async_copy(ent_h.at[idx_t], tbufs[2], sem),
                pltpu.async_copy(ent_t.at[idx_h], tbufs[3], sem),
                pltpu.async_copy(rel_f.at[idx_r], rf_v, sem),
                pltpu.async_copy(rel_i.at[idx_r], ri_v, sem),
            ]
            for cp in cps:
                cp.wait()

            def s_body(e, _):
                a = None
                for q in (0, 1):
                    sl = pl.ds(q * L, L)
                    v = (tbufs[0][e, sl] * rf_v[e, sl] * tbufs[1][e, sl]
                         + tbufs[2][e, sl] * ri_v[e, sl] * tbufs[3][e, sl])
                    a = v if a is None else a + v
                acc[pl.ds(e * L, L)] = a
                return _

            lax.fori_loop(0, C, s_body, None)

            for ia, ib, rel_v in ((idx_h, idx_t, rf_v), (idx_t, idx_h, ri_v)):
                cps = [pltpu.async_copy(tab.at[ia], tbufs[k], sem)
                       for k, tab in enumerate(h_tables)]
                cps += [pltpu.async_copy(tab.at[ib], tbufs[9 + k], sem)
                        for k, tab in enumerate(t_tables)]
                for cp in cps:
                    cp.wait()

                def t_body(e, _):
                    tvs = (ty[e, :], tm[e, :], td[e, :])
                    a = acc[pl.ds(e * L, L)]
                    for q in (0, 1):
                        sl = pl.ds(q * L, L)
                        th = None
                        tt = None
                        for p in range(3):
                            tv = tvs[p]
                            vh = tbufs[3 * p + 2][e, sl] * _sinp(
                                tbufs[3 * p + 0][e, sl] * tv
                                + tbufs[3 * p + 1][e, sl])
                            vt = tbufs[9 + 3 * p + 2][e, sl] * _sinp(
                                tbufs[9 + 3 * p + 0][e, sl] * tv
                                + tbufs[9 + 3 * p + 1][e, sl])
                            th = vh if th is None else th + vh
                            tt = vt if tt is None else tt + vt
                        rT = rel_v[e, pl.ds(SD + q * L, L)]
                        a = a + th * rT * tt
                    acc[pl.ds(e * L, L)] = a
                    return _

                lax.fori_loop(0, C, t_body, None)

            def fin_group(g, _):
                ov = jnp.zeros((L,), jnp.float32)
                for l in range(L):
                    v = acc[pl.ds((g * L + l) * L, L)]
                    h = v + lax.rev(v, (0,))
                    s = h[0]
                    for q in range(1, L // 2):
                        s = s + h[q]
                    ov = jnp.where(rows0 == l, s, ov)
                out_v[pl.ds(g * L, L)] = ov * jnp.float32(0.5)
                return _

            lax.fori_loop(0, C // L, fin_group, None)
            pltpu.sync_copy(out_v, out.at[pl.ds(base, C)])
            return carry

        lax.fori_loop(0, PER_W // C, chunk_body, None)

    return sck


_sck = _make_sc_kernel()


def kernel(heads, rels, tails, years, months, days,
           ent_h, ent_t, rel_f, rel_i,
           y_freq_h, y_freq_t, m_freq_h, m_freq_t, d_freq_h, d_freq_t,
           y_phi_h, y_phi_t, m_phi_h, m_phi_t, d_phi_h, d_phi_t,
           y_amp_h, y_amp_t, m_amp_h, m_amp_t, d_amp_h, d_amp_t):
    years_b = jnp.broadcast_to(years.reshape(-1, 1).astype(jnp.float32), (B, L))
    months_b = jnp.broadcast_to(months.reshape(-1, 1).astype(jnp.float32), (B, L))
    days_b = jnp.broadcast_to(days.reshape(-1, 1).astype(jnp.float32), (B, L))
    return _sck(
        heads.astype(jnp.int32), rels.astype(jnp.int32),
        tails.astype(jnp.int32),
        years_b, months_b, days_b, ent_h, ent_t, rel_f, rel_i,
        y_freq_h, y_freq_t, m_freq_h, m_freq_t, d_freq_h, d_freq_t,
        y_phi_h, y_phi_t, m_phi_h, m_phi_t, d_phi_h, d_phi_t,
        y_amp_h, y_amp_t, m_amp_h, m_amp_t, d_amp_h, d_amp_t)
